# Initial kernel scaffold; baseline (speedup 1.0000x reference)
import jax
import jax.numpy as jnp
from jax.experimental import pallas as pl


def kernel(p1, p3, pair_i, pair_j, basis, diff, W_pp, b_pp, W_pi, W_ii, W_eq_pp, W_pix, W_out, b_out):
    raise NotImplementedError("write your pallas kernel here")


# speedup vs baseline: 8.4868x; 8.4868x over previous
import functools

import jax
import jax.numpy as jnp
from jax import lax
from jax.experimental import pallas as pl
from jax.experimental.pallas import tpu as pltpu
from jax.experimental.pallas import tpu_sc as plsc


def _make_gather(N, E, C):
    NW = 32
    EW = E // NW
    BE = 40
    NB = EW // BE
    mesh = plsc.VectorSubcoreMesh(core_axis_name="c", subcore_axis_name="s")

    @functools.partial(
        pl.kernel,
        out_type=jax.ShapeDtypeStruct((E, C), jnp.float32),
        mesh=mesh,
        scratch_types=[
            pltpu.VMEM((BE,), jnp.int32),
            pltpu.VMEM((BE,), jnp.int32),
            pltpu.VMEM((BE, C), jnp.float32),
            pltpu.VMEM((BE, C), jnp.float32),
            pltpu.SemaphoreType.DMA,
            pltpu.SemaphoreType.DMA,
        ],
    )
    def gather_k(cat_hbm, pi_hbm, pj_hbm, s_hbm, idx_i, idx_j, rows_i, rows_j,
                 sem_i, sem_j):
        cid = lax.axis_index("c")
        sid = lax.axis_index("s")
        wid = sid * 2 + cid
        base0 = wid * EW

        def blk(b, carry):
            base = base0 + b * BE
            pltpu.sync_copy(pi_hbm.at[pl.ds(base, BE)], idx_i)
            pltpu.sync_copy(pj_hbm.at[pl.ds(base, BE)], idx_j)
            ci = pltpu.async_copy(cat_hbm.at[idx_i], rows_i, sem_i)
            cj = pltpu.async_copy(cat_hbm.at[idx_j], rows_j, sem_j)
            ci.wait()
            cj.wait()

            def add_row(e, c2):
                for g in range(C // 16):
                    sl = pl.ds(g * 16, 16)
                    rows_i[e, sl] = rows_i[e, sl] + rows_j[e, sl]
                return c2

            lax.fori_loop(0, BE, add_row, 0)
            pltpu.sync_copy(rows_i, s_hbm.at[pl.ds(base, BE)])
            return carry

        lax.fori_loop(0, NB, blk, 0)

    return gather_k


def _make_edge(E, F, B):
    Eb = 640
    grid = E // Eb
    C = 4 * F

    def body(s_ref, basis_ref, diff_ref, wpi_ref, wii_ref, wpix_ref,
             i1_ref, ix_ref):
        s1 = s_ref[:, :F]
        inter = jnp.tanh(
            jnp.dot(s1, wpi_ref[...], preferred_element_type=jnp.float32))
        u = inter[:, 0:F] * basis_ref[:, 0:1]
        for b in range(1, B):
            u = u + inter[:, b * F:(b + 1) * F] * basis_ref[:, b:b + 1]
        i1 = jnp.tanh(
            jnp.dot(u, wii_ref[...], preferred_element_type=jnp.float32))
        i1_ref[:, 0, :] = i1
        for x in range(3):
            sx = s_ref[:, (x + 1) * F:(x + 2) * F]
            t = jnp.dot(sx, wpix_ref[...], preferred_element_type=jnp.float32)
            ix_ref[:, x, :] = (t + diff_ref[:, x:x + 1]) * i1

    return pl.pallas_call(
        body,
        grid=(grid,),
        in_specs=[
            pl.BlockSpec((Eb, C), lambda i: (i, 0)),
            pl.BlockSpec((Eb, B), lambda i: (i, 0)),
            pl.BlockSpec((Eb, 3), lambda i: (i, 0)),
            pl.BlockSpec((F, F * B), lambda i: (0, 0)),
            pl.BlockSpec((F, F), lambda i: (0, 0)),
            pl.BlockSpec((F, F), lambda i: (0, 0)),
        ],
        out_specs=[
            pl.BlockSpec((Eb, 1, F), lambda i: (i, 0, 0)),
            pl.BlockSpec((Eb, 3, F), lambda i: (i, 0, 0)),
        ],
        out_shape=[
            jax.ShapeDtypeStruct((E, 1, F), jnp.float32),
            jax.ShapeDtypeStruct((E, 3, F), jnp.float32),
        ],
    )


def _make_scatter(N, E, F):
    ET = E // 16
    BE = 80
    NB = ET // BE
    NP = 80
    NPc = N // NP
    mesh = plsc.VectorSubcoreMesh(core_axis_name="c", subcore_axis_name="s")

    @functools.partial(
        pl.kernel,
        out_type=[
            jax.ShapeDtypeStruct((N, F), jnp.float32),
            jax.ShapeDtypeStruct((N, 3 * F), jnp.float32),
        ],
        mesh=mesh,
        scratch_types=[
            pltpu.VMEM((BE,), jnp.int32),
            pltpu.VMEM((BE, F), jnp.float32),
            pltpu.VMEM((NP, F), jnp.float32),
            pltpu.VMEM((NP, F), jnp.float32),
            pltpu.VMEM_SHARED((N, F), jnp.float32),
        ],
    )
    def scatter_k(i1_hbm, ix_hbm, pairi_hbm, zeros_hbm, out1_hbm, out3_hbm,
                  idx_v, rows_v, zbuf, wbuf, acc_sh):
        cid = lax.axis_index("c")
        sid = lax.axis_index("s")
        pltpu.sync_copy(zeros_hbm, zbuf)

        npieces = (NPc - sid + 15) // 16

        def run_pass(src_at, dst_at):
            def zero_piece(k, carry):
                r0 = (sid + 16 * k) * NP
                pltpu.sync_copy(zbuf, acc_sh.at[pl.ds(r0, NP)])
                return carry

            lax.fori_loop(0, npieces, zero_piece, 0)
            plsc.subcore_barrier()

            def blk(b, carry):
                base = sid * ET + b * BE
                pltpu.sync_copy(pairi_hbm.at[pl.ds(base, BE)], idx_v)
                pltpu.sync_copy(src_at(base), rows_v)
                pltpu.sync_copy(rows_v, acc_sh.at[idx_v], add=True)
                return carry

            lax.fori_loop(0, NB, blk, 0)
            plsc.subcore_barrier()

            def write_piece(k, carry):
                r0 = (sid + 16 * k) * NP
                pltpu.sync_copy(acc_sh.at[pl.ds(r0, NP)], wbuf)
                pltpu.sync_copy(wbuf, dst_at(r0))
                return carry

            lax.fori_loop(0, npieces, write_piece, 0)

        @pl.when(cid == 0)
        def _():
            run_pass(lambda b: i1_hbm.at[pl.ds(b, BE)],
                     lambda r: out1_hbm.at[pl.ds(r, NP)])
            run_pass(lambda b: ix_hbm.at[pl.ds(b, BE), pl.ds(0, F)],
                     lambda r: out3_hbm.at[pl.ds(r, NP), pl.ds(0, F)])

        @pl.when(cid == 1)
        def _():
            run_pass(lambda b: ix_hbm.at[pl.ds(b, BE), pl.ds(F, F)],
                     lambda r: out3_hbm.at[pl.ds(r, NP), pl.ds(F, F)])
            run_pass(lambda b: ix_hbm.at[pl.ds(b, BE), pl.ds(2 * F, F)],
                     lambda r: out3_hbm.at[pl.ds(r, NP), pl.ds(2 * F, F)])

    return scatter_k


def _make_node(N, F):
    Nb = 2000
    grid = N // Nb

    def body(a1_ref, a3_ref, wpp_ref, bpp_ref, weq_ref, wout_ref, bout_ref,
             p1t1_ref, p3t1_ref):
        p1n = jnp.tanh(
            jnp.dot(a1_ref[...], wpp_ref[...],
                    preferred_element_type=jnp.float32) + bpp_ref[...])
        p1t1_ref[:, 0, :] = jnp.dot(
            p1n, wout_ref[...], preferred_element_type=jnp.float32) + bout_ref[...]
        p3n = [
            jnp.dot(a3_ref[:, x * F:(x + 1) * F], weq_ref[...],
                    preferred_element_type=jnp.float32) for x in range(3)
        ]
        dot = p3n[0] * p3n[0] + p3n[1] * p3n[1] + p3n[2] * p3n[2]
        scale = jnp.dot(
            dot, wout_ref[...], preferred_element_type=jnp.float32) + bout_ref[...]
        for x in range(3):
            p3t1_ref[:, x, :] = p3n[x] * scale

    return pl.pallas_call(
        body,
        grid=(grid,),
        in_specs=[
            pl.BlockSpec((Nb, F), lambda i: (i, 0)),
            pl.BlockSpec((Nb, 3 * F), lambda i: (i, 0)),
            pl.BlockSpec((F, F), lambda i: (0, 0)),
            pl.BlockSpec((1, F), lambda i: (0, 0)),
            pl.BlockSpec((F, F), lambda i: (0, 0)),
            pl.BlockSpec((F, F), lambda i: (0, 0)),
            pl.BlockSpec((1, F), lambda i: (0, 0)),
        ],
        out_specs=[
            pl.BlockSpec((Nb, 1, F), lambda i: (i, 0, 0)),
            pl.BlockSpec((Nb, 3, F), lambda i: (i, 0, 0)),
        ],
        out_shape=[
            jax.ShapeDtypeStruct((N, 1, F), jnp.float32),
            jax.ShapeDtypeStruct((N, 3, F), jnp.float32),
        ],
    )


def kernel(p1, p3, pair_i, pair_j, basis, diff, W_pp, b_pp, W_pi, W_ii,
           W_eq_pp, W_pix, W_out, b_out):
    N, _, F = p1.shape
    E = pair_i.shape[0]
    B = basis.shape[1]

    cat = jnp.concatenate([p1.reshape(N, F), p3.reshape(N, 3 * F)], axis=1)
    s = _make_gather(N, E, 4 * F)(cat, pair_i, pair_j)

    W_pi_perm = W_pi.reshape(F, F, B).transpose(0, 2, 1).reshape(F, F * B)
    i1, ix = _make_edge(E, F, B)(s, basis, diff, W_pi_perm, W_ii, W_pix)

    zeros = jnp.zeros((80, F), jnp.float32)
    acc1, acc3 = _make_scatter(N, E, F)(
        i1.reshape(E, F), ix.reshape(E, 3 * F), pair_i, zeros)

    p1t1, p3t1 = _make_node(N, F)(
        acc1, acc3, W_pp, b_pp.reshape(1, F), W_eq_pp, W_out,
        b_out.reshape(1, F))
    return (p1t1, p3t1, i1, ix)


# speedup vs baseline: 10.6828x; 1.2588x over previous
import functools

import jax
import jax.numpy as jnp
from jax import lax
from jax.experimental import pallas as pl
from jax.experimental.pallas import tpu as pltpu
from jax.experimental.pallas import tpu_sc as plsc


def _make_gather(N, E, C):
    NW = 32
    EW = E // NW
    BE = 40
    NB = EW // BE
    mesh = plsc.VectorSubcoreMesh(core_axis_name="c", subcore_axis_name="s")

    @functools.partial(
        pl.kernel,
        out_type=jax.ShapeDtypeStruct((E, C), jnp.float32),
        mesh=mesh,
        scratch_types=[
            pltpu.VMEM((BE,), jnp.int32),
            pltpu.VMEM((BE,), jnp.int32),
            pltpu.VMEM((BE, C), jnp.float32),
            pltpu.VMEM((BE, C), jnp.float32),
            pltpu.SemaphoreType.DMA,
            pltpu.SemaphoreType.DMA,
        ],
    )
    def gather_k(cat_hbm, pi_hbm, pj_hbm, s_hbm, idx_i, idx_j, rows_i, rows_j,
                 sem_i, sem_j):
        cid = lax.axis_index("c")
        sid = lax.axis_index("s")
        wid = sid * 2 + cid
        base0 = wid * EW

        def blk(b, carry):
            base = base0 + b * BE
            pltpu.sync_copy(pi_hbm.at[pl.ds(base, BE)], idx_i)
            pltpu.sync_copy(pj_hbm.at[pl.ds(base, BE)], idx_j)
            ci = pltpu.async_copy(cat_hbm.at[idx_i], rows_i, sem_i)
            cj = pltpu.async_copy(cat_hbm.at[idx_j], rows_j, sem_j)
            ci.wait()
            cj.wait()

            def add_row(e, c2):
                for g in range(C // 16):
                    sl = pl.ds(g * 16, 16)
                    rows_i[e, sl] = rows_i[e, sl] + rows_j[e, sl]
                return c2

            lax.fori_loop(0, BE, add_row, 0)
            pltpu.sync_copy(rows_i, s_hbm.at[pl.ds(base, BE)])
            return carry

        lax.fori_loop(0, NB, blk, 0)

    return gather_k


def _make_edge(E, F, B):
    Eb = 640
    grid = E // Eb
    C = 4 * F

    def body(s_ref, basis_ref, diff_ref, wpi_ref, wii_ref, wpix_ref,
             i1_ref, ix_ref):
        s1 = s_ref[:, :F]
        inter = jnp.tanh(
            jnp.dot(s1, wpi_ref[...], preferred_element_type=jnp.float32))
        u = inter[:, 0:F] * basis_ref[:, 0:1]
        for b in range(1, B):
            u = u + inter[:, b * F:(b + 1) * F] * basis_ref[:, b:b + 1]
        i1 = jnp.tanh(
            jnp.dot(u, wii_ref[...], preferred_element_type=jnp.float32))
        i1_ref[...] = i1
        for x in range(3):
            sx = s_ref[:, (x + 1) * F:(x + 2) * F]
            t = jnp.dot(sx, wpix_ref[...], preferred_element_type=jnp.float32)
            ix_ref[:, x * F:(x + 1) * F] = (t + diff_ref[:, x:x + 1]) * i1

    return pl.pallas_call(
        body,
        grid=(grid,),
        in_specs=[
            pl.BlockSpec((Eb, C), lambda i: (i, 0)),
            pl.BlockSpec((Eb, B), lambda i: (i, 0)),
            pl.BlockSpec((Eb, 3), lambda i: (i, 0)),
            pl.BlockSpec((F, F * B), lambda i: (0, 0)),
            pl.BlockSpec((F, F), lambda i: (0, 0)),
            pl.BlockSpec((F, F), lambda i: (0, 0)),
        ],
        out_specs=[
            pl.BlockSpec((Eb, F), lambda i: (i, 0)),
            pl.BlockSpec((Eb, 3 * F), lambda i: (i, 0)),
        ],
        out_shape=[
            jax.ShapeDtypeStruct((E, F), jnp.float32),
            jax.ShapeDtypeStruct((E, 3 * F), jnp.float32),
        ],
    )


def _make_scatter(N, E, F):
    ET = E // 16
    BE = 80
    NB = ET // BE
    NP = 80
    NPc = N // NP
    mesh = plsc.VectorSubcoreMesh(core_axis_name="c", subcore_axis_name="s")

    @functools.partial(
        pl.kernel,
        out_type=[
            jax.ShapeDtypeStruct((N, F), jnp.float32),
            jax.ShapeDtypeStruct((N, 3 * F), jnp.float32),
        ],
        mesh=mesh,
        scratch_types=[
            pltpu.VMEM((BE,), jnp.int32),
            pltpu.VMEM((BE, F), jnp.float32),
            pltpu.VMEM((NP, F), jnp.float32),
            pltpu.VMEM((NP, F), jnp.float32),
            pltpu.VMEM_SHARED((N, F), jnp.float32),
        ],
    )
    def scatter_k(i1_hbm, ix_hbm, pairi_hbm, zeros_hbm, out1_hbm, out3_hbm,
                  idx_v, rows_v, zbuf, wbuf, acc_sh):
        cid = lax.axis_index("c")
        sid = lax.axis_index("s")
        pltpu.sync_copy(zeros_hbm, zbuf)

        npieces = (NPc - sid + 15) // 16

        def run_pass(src_at, dst_at):
            def zero_piece(k, carry):
                r0 = (sid + 16 * k) * NP
                pltpu.sync_copy(zbuf, acc_sh.at[pl.ds(r0, NP)])
                return carry

            lax.fori_loop(0, npieces, zero_piece, 0)
            plsc.subcore_barrier()

            def blk(b, carry):
                base = sid * ET + b * BE
                pltpu.sync_copy(pairi_hbm.at[pl.ds(base, BE)], idx_v)
                pltpu.sync_copy(src_at(base), rows_v)
                pltpu.sync_copy(rows_v, acc_sh.at[idx_v], add=True)
                return carry

            lax.fori_loop(0, NB, blk, 0)
            plsc.subcore_barrier()

            def write_piece(k, carry):
                r0 = (sid + 16 * k) * NP
                pltpu.sync_copy(acc_sh.at[pl.ds(r0, NP)], wbuf)
                pltpu.sync_copy(wbuf, dst_at(r0))
                return carry

            lax.fori_loop(0, npieces, write_piece, 0)

        @pl.when(cid == 0)
        def _():
            run_pass(lambda b: i1_hbm.at[pl.ds(b, BE)],
                     lambda r: out1_hbm.at[pl.ds(r, NP)])
            run_pass(lambda b: ix_hbm.at[pl.ds(b, BE), pl.ds(0, F)],
                     lambda r: out3_hbm.at[pl.ds(r, NP), pl.ds(0, F)])

        @pl.when(cid == 1)
        def _():
            run_pass(lambda b: ix_hbm.at[pl.ds(b, BE), pl.ds(F, F)],
                     lambda r: out3_hbm.at[pl.ds(r, NP), pl.ds(F, F)])
            run_pass(lambda b: ix_hbm.at[pl.ds(b, BE), pl.ds(2 * F, F)],
                     lambda r: out3_hbm.at[pl.ds(r, NP), pl.ds(2 * F, F)])

    return scatter_k


def _make_node(N, F):
    Nb = 2000
    grid = N // Nb

    def body(a1_ref, a3_ref, wpp_ref, bpp_ref, weq_ref, wout_ref, bout_ref,
             p1t1_ref, p3t1_ref):
        p1n = jnp.tanh(
            jnp.dot(a1_ref[...], wpp_ref[...],
                    preferred_element_type=jnp.float32) + bpp_ref[...])
        p1t1_ref[:, 0, :] = jnp.dot(
            p1n, wout_ref[...], preferred_element_type=jnp.float32) + bout_ref[...]
        p3n = [
            jnp.dot(a3_ref[:, x * F:(x + 1) * F], weq_ref[...],
                    preferred_element_type=jnp.float32) for x in range(3)
        ]
        dot = p3n[0] * p3n[0] + p3n[1] * p3n[1] + p3n[2] * p3n[2]
        scale = jnp.dot(
            dot, wout_ref[...], preferred_element_type=jnp.float32) + bout_ref[...]
        for x in range(3):
            p3t1_ref[:, x, :] = p3n[x] * scale

    return pl.pallas_call(
        body,
        grid=(grid,),
        in_specs=[
            pl.BlockSpec((Nb, F), lambda i: (i, 0)),
            pl.BlockSpec((Nb, 3 * F), lambda i: (i, 0)),
            pl.BlockSpec((F, F), lambda i: (0, 0)),
            pl.BlockSpec((1, F), lambda i: (0, 0)),
            pl.BlockSpec((F, F), lambda i: (0, 0)),
            pl.BlockSpec((F, F), lambda i: (0, 0)),
            pl.BlockSpec((1, F), lambda i: (0, 0)),
        ],
        out_specs=[
            pl.BlockSpec((Nb, 1, F), lambda i: (i, 0, 0)),
            pl.BlockSpec((Nb, 3, F), lambda i: (i, 0, 0)),
        ],
        out_shape=[
            jax.ShapeDtypeStruct((N, 1, F), jnp.float32),
            jax.ShapeDtypeStruct((N, 3, F), jnp.float32),
        ],
    )


def kernel(p1, p3, pair_i, pair_j, basis, diff, W_pp, b_pp, W_pi, W_ii,
           W_eq_pp, W_pix, W_out, b_out):
    N, _, F = p1.shape
    E = pair_i.shape[0]
    B = basis.shape[1]

    cat = jnp.concatenate([p1.reshape(N, F), p3.reshape(N, 3 * F)], axis=1)
    s = _make_gather(N, E, 4 * F)(cat, pair_i, pair_j)

    W_pi_perm = W_pi.reshape(F, F, B).transpose(0, 2, 1).reshape(F, F * B)
    i1f, ixf = _make_edge(E, F, B)(s, basis, diff, W_pi_perm, W_ii, W_pix)

    zeros = jnp.zeros((80, F), jnp.float32)
    acc1, acc3 = _make_scatter(N, E, F)(i1f, ixf, pair_i, zeros)

    p1t1, p3t1 = _make_node(N, F)(
        acc1, acc3, W_pp, b_pp.reshape(1, F), W_eq_pp, W_out,
        b_out.reshape(1, F))
    return (p1t1, p3t1, i1f.reshape(E, 1, F), ixf.reshape(E, 3, F))


# speedup vs baseline: 12.7277x; 1.1914x over previous
import functools

import jax
import jax.numpy as jnp
from jax import lax
from jax.experimental import pallas as pl
from jax.experimental.pallas import tpu as pltpu
from jax.experimental.pallas import tpu_sc as plsc


def _make_gather(N, E, C):
    NW = 32
    EW = E // NW
    BE = 40
    NB = EW // BE
    mesh = plsc.VectorSubcoreMesh(core_axis_name="c", subcore_axis_name="s")

    @functools.partial(
        pl.kernel,
        out_type=jax.ShapeDtypeStruct((E, C), jnp.float32),
        mesh=mesh,
        scratch_types=[
            pltpu.VMEM((EW,), jnp.int32),
            pltpu.VMEM((EW,), jnp.int32),
            pltpu.VMEM((BE, C), jnp.float32),
            pltpu.VMEM((BE, C), jnp.float32),
            pltpu.VMEM((BE, C), jnp.float32),
            pltpu.VMEM((BE, C), jnp.float32),
            pltpu.SemaphoreType.DMA,
            pltpu.SemaphoreType.DMA,
            pltpu.SemaphoreType.DMA,
            pltpu.SemaphoreType.DMA,
        ],
    )
    def gather_k(cat_hbm, pi_hbm, pj_hbm, s_hbm, idx_ia, idx_ja,
                 ri0, rj0, ri1, rj1, si0, sj0, si1, sj1):
        cid = lax.axis_index("c")
        sid = lax.axis_index("s")
        wid = sid * 2 + cid
        base0 = wid * EW
        pltpu.sync_copy(pi_hbm.at[pl.ds(base0, EW)], idx_ia)
        pltpu.sync_copy(pj_hbm.at[pl.ds(base0, EW)], idx_ja)

        def fire(b, ri, rj, si, sj):
            pltpu.async_copy(cat_hbm.at[idx_ia.at[pl.ds(b * BE, BE)]], ri, si)
            pltpu.async_copy(cat_hbm.at[idx_ja.at[pl.ds(b * BE, BE)]], rj, sj)

        def finish(b, ri, rj, si, sj):
            pltpu.make_async_copy(
                cat_hbm.at[idx_ia.at[pl.ds(b * BE, BE)]], ri, si).wait()
            pltpu.make_async_copy(
                cat_hbm.at[idx_ja.at[pl.ds(b * BE, BE)]], rj, sj).wait()

            def add_row(e, c2):
                for g in range(C // 16):
                    sl = pl.ds(g * 16, 16)
                    ri[e, sl] = ri[e, sl] + rj[e, sl]
                return c2

            lax.fori_loop(0, BE, add_row, 0)
            pltpu.sync_copy(ri, s_hbm.at[pl.ds(base0 + b * BE, BE)])

        fire(0, ri0, rj0, si0, sj0)

        def body(b2, carry):
            b0 = 2 * b2
            fire(b0 + 1, ri1, rj1, si1, sj1)
            finish(b0, ri0, rj0, si0, sj0)
            fire(b0 + 2, ri0, rj0, si0, sj0)
            finish(b0 + 1, ri1, rj1, si1, sj1)
            return carry

        lax.fori_loop(0, (NB - 1) // 2, body, 0)
        finish(NB - 1, ri0, rj0, si0, sj0)

    return gather_k


def _make_edge(E, F, B):
    Eb = 640
    grid = E // Eb
    C = 4 * F

    def body(s_ref, basis_ref, diff_ref, wpi_ref, wii_ref, wpix_ref,
             i1_ref, ix_ref):
        s1 = s_ref[:, :F]
        inter = jnp.tanh(
            jnp.dot(s1, wpi_ref[...], preferred_element_type=jnp.float32))
        u = inter[:, 0:F] * basis_ref[:, 0:1]
        for b in range(1, B):
            u = u + inter[:, b * F:(b + 1) * F] * basis_ref[:, b:b + 1]
        i1 = jnp.tanh(
            jnp.dot(u, wii_ref[...], preferred_element_type=jnp.float32))
        i1_ref[...] = i1
        for x in range(3):
            sx = s_ref[:, (x + 1) * F:(x + 2) * F]
            t = jnp.dot(sx, wpix_ref[...], preferred_element_type=jnp.float32)
            ix_ref[:, x * F:(x + 1) * F] = (t + diff_ref[:, x:x + 1]) * i1

    return pl.pallas_call(
        body,
        grid=(grid,),
        in_specs=[
            pl.BlockSpec((Eb, C), lambda i: (i, 0)),
            pl.BlockSpec((Eb, B), lambda i: (i, 0)),
            pl.BlockSpec((Eb, 3), lambda i: (i, 0)),
            pl.BlockSpec((F, F * B), lambda i: (0, 0)),
            pl.BlockSpec((F, F), lambda i: (0, 0)),
            pl.BlockSpec((F, F), lambda i: (0, 0)),
        ],
        out_specs=[
            pl.BlockSpec((Eb, F), lambda i: (i, 0)),
            pl.BlockSpec((Eb, 3 * F), lambda i: (i, 0)),
        ],
        out_shape=[
            jax.ShapeDtypeStruct((E, F), jnp.float32),
            jax.ShapeDtypeStruct((E, 3 * F), jnp.float32),
        ],
    )


def _make_scatter(N, E, F):
    ET = E // 16
    BE = 80
    NB = ET // BE
    NP = 80
    NPc = N // NP
    mesh = plsc.VectorSubcoreMesh(core_axis_name="c", subcore_axis_name="s")

    @functools.partial(
        pl.kernel,
        out_type=[
            jax.ShapeDtypeStruct((N, F), jnp.float32),
            jax.ShapeDtypeStruct((N, 3 * F), jnp.float32),
        ],
        mesh=mesh,
        scratch_types=[
            pltpu.VMEM((BE,), jnp.int32),
            pltpu.VMEM((BE, F), jnp.float32),
            pltpu.VMEM((NP, F), jnp.float32),
            pltpu.VMEM((NP, F), jnp.float32),
            pltpu.VMEM_SHARED((N, F), jnp.float32),
        ],
    )
    def scatter_k(i1_hbm, ix_hbm, pairi_hbm, zeros_hbm, out1_hbm, out3_hbm,
                  idx_v, rows_v, zbuf, wbuf, acc_sh):
        cid = lax.axis_index("c")
        sid = lax.axis_index("s")
        pltpu.sync_copy(zeros_hbm, zbuf)

        npieces = (NPc - sid + 15) // 16

        def run_pass(src_at, dst_at):
            def zero_piece(k, carry):
                r0 = (sid + 16 * k) * NP
                pltpu.sync_copy(zbuf, acc_sh.at[pl.ds(r0, NP)])
                return carry

            lax.fori_loop(0, npieces, zero_piece, 0)
            plsc.subcore_barrier()

            def blk(b, carry):
                base = sid * ET + b * BE
                pltpu.sync_copy(pairi_hbm.at[pl.ds(base, BE)], idx_v)
                pltpu.sync_copy(src_at(base), rows_v)
                pltpu.sync_copy(rows_v, acc_sh.at[idx_v], add=True)
                return carry

            lax.fori_loop(0, NB, blk, 0)
            plsc.subcore_barrier()

            def write_piece(k, carry):
                r0 = (sid + 16 * k) * NP
                pltpu.sync_copy(acc_sh.at[pl.ds(r0, NP)], wbuf)
                pltpu.sync_copy(wbuf, dst_at(r0))
                return carry

            lax.fori_loop(0, npieces, write_piece, 0)

        @pl.when(cid == 0)
        def _():
            run_pass(lambda b: i1_hbm.at[pl.ds(b, BE)],
                     lambda r: out1_hbm.at[pl.ds(r, NP)])
            run_pass(lambda b: ix_hbm.at[pl.ds(b, BE), pl.ds(0, F)],
                     lambda r: out3_hbm.at[pl.ds(r, NP), pl.ds(0, F)])

        @pl.when(cid == 1)
        def _():
            run_pass(lambda b: ix_hbm.at[pl.ds(b, BE), pl.ds(F, F)],
                     lambda r: out3_hbm.at[pl.ds(r, NP), pl.ds(F, F)])
            run_pass(lambda b: ix_hbm.at[pl.ds(b, BE), pl.ds(2 * F, F)],
                     lambda r: out3_hbm.at[pl.ds(r, NP), pl.ds(2 * F, F)])

    return scatter_k


def _make_node(N, F):
    Nb = 2000
    grid = N // Nb

    def body(a1_ref, a3_ref, wpp_ref, bpp_ref, weq_ref, wout_ref, bout_ref,
             p1t1_ref, p3t1_ref):
        p1n = jnp.tanh(
            jnp.dot(a1_ref[...], wpp_ref[...],
                    preferred_element_type=jnp.float32) + bpp_ref[...])
        p1t1_ref[:, 0, :] = jnp.dot(
            p1n, wout_ref[...], preferred_element_type=jnp.float32) + bout_ref[...]
        p3n = [
            jnp.dot(a3_ref[:, x * F:(x + 1) * F], weq_ref[...],
                    preferred_element_type=jnp.float32) for x in range(3)
        ]
        dot = p3n[0] * p3n[0] + p3n[1] * p3n[1] + p3n[2] * p3n[2]
        scale = jnp.dot(
            dot, wout_ref[...], preferred_element_type=jnp.float32) + bout_ref[...]
        for x in range(3):
            p3t1_ref[:, x, :] = p3n[x] * scale

    return pl.pallas_call(
        body,
        grid=(grid,),
        in_specs=[
            pl.BlockSpec((Nb, F), lambda i: (i, 0)),
            pl.BlockSpec((Nb, 3 * F), lambda i: (i, 0)),
            pl.BlockSpec((F, F), lambda i: (0, 0)),
            pl.BlockSpec((1, F), lambda i: (0, 0)),
            pl.BlockSpec((F, F), lambda i: (0, 0)),
            pl.BlockSpec((F, F), lambda i: (0, 0)),
            pl.BlockSpec((1, F), lambda i: (0, 0)),
        ],
        out_specs=[
            pl.BlockSpec((Nb, 1, F), lambda i: (i, 0, 0)),
            pl.BlockSpec((Nb, 3, F), lambda i: (i, 0, 0)),
        ],
        out_shape=[
            jax.ShapeDtypeStruct((N, 1, F), jnp.float32),
            jax.ShapeDtypeStruct((N, 3, F), jnp.float32),
        ],
    )


def kernel(p1, p3, pair_i, pair_j, basis, diff, W_pp, b_pp, W_pi, W_ii,
           W_eq_pp, W_pix, W_out, b_out):
    N, _, F = p1.shape
    E = pair_i.shape[0]
    B = basis.shape[1]

    cat = jnp.concatenate([p1.reshape(N, F), p3.reshape(N, 3 * F)], axis=1)
    s = _make_gather(N, E, 4 * F)(cat, pair_i, pair_j)

    W_pi_perm = W_pi.reshape(F, F, B).transpose(0, 2, 1).reshape(F, F * B)
    i1f, ixf = _make_edge(E, F, B)(s, basis, diff, W_pi_perm, W_ii, W_pix)

    zeros = jnp.zeros((80, F), jnp.float32)
    acc1, acc3 = _make_scatter(N, E, F)(i1f, ixf, pair_i, zeros)

    p1t1, p3t1 = _make_node(N, F)(
        acc1, acc3, W_pp, b_pp.reshape(1, F), W_eq_pp, W_out,
        b_out.reshape(1, F))
    return (p1t1, p3t1, i1f.reshape(E, 1, F), ixf.reshape(E, 3, F))


# speedup vs baseline: 15.1732x; 1.1921x over previous
import functools

import jax
import jax.numpy as jnp
from jax import lax
from jax.experimental import pallas as pl
from jax.experimental.pallas import tpu as pltpu
from jax.experimental.pallas import tpu_sc as plsc


def _make_gather(N, E, C):
    NW = 32
    EW = E // NW
    BE = 40
    NB = EW // BE
    mesh = plsc.VectorSubcoreMesh(core_axis_name="c", subcore_axis_name="s")

    @functools.partial(
        pl.kernel,
        out_type=jax.ShapeDtypeStruct((E, C), jnp.float32),
        mesh=mesh,
        scratch_types=[
            pltpu.VMEM((EW,), jnp.int32),
            pltpu.VMEM((EW,), jnp.int32),
            pltpu.VMEM((BE, C), jnp.float32),
            pltpu.VMEM((BE, C), jnp.float32),
            pltpu.VMEM((BE, C), jnp.float32),
            pltpu.VMEM((BE, C), jnp.float32),
            pltpu.SemaphoreType.DMA,
            pltpu.SemaphoreType.DMA,
            pltpu.SemaphoreType.DMA,
            pltpu.SemaphoreType.DMA,
        ],
    )
    def gather_k(cat_hbm, pi_hbm, pj_hbm, s_hbm, idx_ia, idx_ja,
                 ri0, rj0, ri1, rj1, si0, sj0, si1, sj1):
        cid = lax.axis_index("c")
        sid = lax.axis_index("s")
        wid = sid * 2 + cid
        base0 = wid * EW
        pltpu.sync_copy(pi_hbm.at[pl.ds(base0, EW)], idx_ia)
        pltpu.sync_copy(pj_hbm.at[pl.ds(base0, EW)], idx_ja)

        def fire(b, ri, rj, si, sj):
            pltpu.async_copy(cat_hbm.at[idx_ia.at[pl.ds(b * BE, BE)]], ri, si)
            pltpu.async_copy(cat_hbm.at[idx_ja.at[pl.ds(b * BE, BE)]], rj, sj)

        def finish(b, ri, rj, si, sj):
            pltpu.make_async_copy(
                cat_hbm.at[idx_ia.at[pl.ds(b * BE, BE)]], ri, si).wait()
            pltpu.make_async_copy(
                cat_hbm.at[idx_ja.at[pl.ds(b * BE, BE)]], rj, sj).wait()

            def add_row(e, c2):
                for g in range(C // 16):
                    sl = pl.ds(g * 16, 16)
                    ri[e, sl] = ri[e, sl] + rj[e, sl]
                return c2

            lax.fori_loop(0, BE, add_row, 0)
            pltpu.sync_copy(ri, s_hbm.at[pl.ds(base0 + b * BE, BE)])

        fire(0, ri0, rj0, si0, sj0)

        def body(b2, carry):
            b0 = 2 * b2
            fire(b0 + 1, ri1, rj1, si1, sj1)
            finish(b0, ri0, rj0, si0, sj0)
            fire(b0 + 2, ri0, rj0, si0, sj0)
            finish(b0 + 1, ri1, rj1, si1, sj1)
            return carry

        lax.fori_loop(0, (NB - 1) // 2, body, 0)
        finish(NB - 1, ri0, rj0, si0, sj0)

    return gather_k


def _make_edge(E, F, B):
    Eb = 640
    grid = E // Eb
    C = 4 * F

    def body(s_ref, basis_ref, diff_ref, wpi_ref, wii_ref, wpix_ref,
             i1_ref, ix_ref):
        s1 = s_ref[:, :F]
        inter = jnp.tanh(
            jnp.dot(s1, wpi_ref[...], preferred_element_type=jnp.float32))
        u = inter[:, 0:F] * basis_ref[:, 0:1]
        for b in range(1, B):
            u = u + inter[:, b * F:(b + 1) * F] * basis_ref[:, b:b + 1]
        i1 = jnp.tanh(
            jnp.dot(u, wii_ref[...], preferred_element_type=jnp.float32))
        i1_ref[...] = i1
        for x in range(3):
            sx = s_ref[:, (x + 1) * F:(x + 2) * F]
            t = jnp.dot(sx, wpix_ref[...], preferred_element_type=jnp.float32)
            ix_ref[:, x * F:(x + 1) * F] = (t + diff_ref[:, x:x + 1]) * i1

    return pl.pallas_call(
        body,
        grid=(grid,),
        in_specs=[
            pl.BlockSpec((Eb, C), lambda i: (i, 0)),
            pl.BlockSpec((Eb, B), lambda i: (i, 0)),
            pl.BlockSpec((Eb, 3), lambda i: (i, 0)),
            pl.BlockSpec((F, F * B), lambda i: (0, 0)),
            pl.BlockSpec((F, F), lambda i: (0, 0)),
            pl.BlockSpec((F, F), lambda i: (0, 0)),
        ],
        out_specs=[
            pl.BlockSpec((Eb, F), lambda i: (i, 0)),
            pl.BlockSpec((Eb, 3 * F), lambda i: (i, 0)),
        ],
        out_shape=[
            jax.ShapeDtypeStruct((E, F), jnp.float32),
            jax.ShapeDtypeStruct((E, 3 * F), jnp.float32),
        ],
    )


def _make_scatter(N, E, F):
    ET = E // 16
    BE = 80
    NB = ET // BE
    NP = 80
    NPc = N // NP
    mesh = plsc.VectorSubcoreMesh(core_axis_name="c", subcore_axis_name="s")

    @functools.partial(
        pl.kernel,
        out_type=[
            jax.ShapeDtypeStruct((N, F), jnp.float32),
            jax.ShapeDtypeStruct((N, 3 * F), jnp.float32),
        ],
        mesh=mesh,
        scratch_types=[
            pltpu.VMEM((BE,), jnp.int32),
            pltpu.VMEM((BE,), jnp.int32),
            pltpu.VMEM((BE, F), jnp.float32),
            pltpu.VMEM((BE, F), jnp.float32),
            pltpu.VMEM((NP, F), jnp.float32),
            pltpu.VMEM((NP, F), jnp.float32),
            pltpu.VMEM_SHARED((N, F), jnp.float32),
            pltpu.SemaphoreType.DMA,
            pltpu.SemaphoreType.DMA,
            pltpu.SemaphoreType.DMA,
            pltpu.SemaphoreType.DMA,
        ],
    )
    def scatter_k(i1_hbm, ix_hbm, pairi_hbm, zeros_hbm, out1_hbm, out3_hbm,
                  idx0, idx1, r0b, r1b, zbuf, wbuf, acc_sh,
                  sI0, sR0, sI1, sR1):
        cid = lax.axis_index("c")
        sid = lax.axis_index("s")
        pltpu.sync_copy(zeros_hbm, zbuf)

        npieces = (NPc - sid + 15) // 16

        def run_pass(src_at, dst_at):
            def zero_piece(k, carry):
                r0 = (sid + 16 * k) * NP
                pltpu.sync_copy(zbuf, acc_sh.at[pl.ds(r0, NP)])
                return carry

            lax.fori_loop(0, npieces, zero_piece, 0)
            plsc.subcore_barrier()

            def fire(b, idx_v, rows_v, sI, sR):
                base = sid * ET + b * BE
                pltpu.async_copy(pairi_hbm.at[pl.ds(base, BE)], idx_v, sI)
                pltpu.async_copy(src_at(base), rows_v, sR)

            def finish(b, idx_v, rows_v, sI, sR):
                base = sid * ET + b * BE
                pltpu.make_async_copy(
                    pairi_hbm.at[pl.ds(base, BE)], idx_v, sI).wait()
                pltpu.make_async_copy(src_at(base), rows_v, sR).wait()
                pltpu.sync_copy(rows_v, acc_sh.at[idx_v], add=True)

            fire(0, idx0, r0b, sI0, sR0)

            def blk(b2, carry):
                b0 = 2 * b2
                fire(b0 + 1, idx1, r1b, sI1, sR1)
                finish(b0, idx0, r0b, sI0, sR0)
                fire(b0 + 2, idx0, r0b, sI0, sR0)
                finish(b0 + 1, idx1, r1b, sI1, sR1)
                return carry

            lax.fori_loop(0, (NB - 1) // 2, blk, 0)
            finish(NB - 1, idx0, r0b, sI0, sR0)
            plsc.subcore_barrier()

            def write_piece(k, carry):
                r0 = (sid + 16 * k) * NP
                pltpu.sync_copy(acc_sh.at[pl.ds(r0, NP)], wbuf)
                pltpu.sync_copy(wbuf, dst_at(r0))
                return carry

            lax.fori_loop(0, npieces, write_piece, 0)

        @pl.when(cid == 0)
        def _():
            run_pass(lambda b: i1_hbm.at[pl.ds(b, BE)],
                     lambda r: out1_hbm.at[pl.ds(r, NP)])
            run_pass(lambda b: ix_hbm.at[pl.ds(b, BE), pl.ds(0, F)],
                     lambda r: out3_hbm.at[pl.ds(r, NP), pl.ds(0, F)])

        @pl.when(cid == 1)
        def _():
            run_pass(lambda b: ix_hbm.at[pl.ds(b, BE), pl.ds(F, F)],
                     lambda r: out3_hbm.at[pl.ds(r, NP), pl.ds(F, F)])
            run_pass(lambda b: ix_hbm.at[pl.ds(b, BE), pl.ds(2 * F, F)],
                     lambda r: out3_hbm.at[pl.ds(r, NP), pl.ds(2 * F, F)])

    return scatter_k


def _make_node(N, F):
    Nb = 2000
    grid = N // Nb

    def body(a1_ref, a3_ref, wpp_ref, bpp_ref, weq_ref, wout_ref, bout_ref,
             p1t1_ref, p3t1_ref):
        p1n = jnp.tanh(
            jnp.dot(a1_ref[...], wpp_ref[...],
                    preferred_element_type=jnp.float32) + bpp_ref[...])
        p1t1_ref[:, 0, :] = jnp.dot(
            p1n, wout_ref[...], preferred_element_type=jnp.float32) + bout_ref[...]
        p3n = [
            jnp.dot(a3_ref[:, x * F:(x + 1) * F], weq_ref[...],
                    preferred_element_type=jnp.float32) for x in range(3)
        ]
        dot = p3n[0] * p3n[0] + p3n[1] * p3n[1] + p3n[2] * p3n[2]
        scale = jnp.dot(
            dot, wout_ref[...], preferred_element_type=jnp.float32) + bout_ref[...]
        for x in range(3):
            p3t1_ref[:, x, :] = p3n[x] * scale

    return pl.pallas_call(
        body,
        grid=(grid,),
        in_specs=[
            pl.BlockSpec((Nb, F), lambda i: (i, 0)),
            pl.BlockSpec((Nb, 3 * F), lambda i: (i, 0)),
            pl.BlockSpec((F, F), lambda i: (0, 0)),
            pl.BlockSpec((1, F), lambda i: (0, 0)),
            pl.BlockSpec((F, F), lambda i: (0, 0)),
            pl.BlockSpec((F, F), lambda i: (0, 0)),
            pl.BlockSpec((1, F), lambda i: (0, 0)),
        ],
        out_specs=[
            pl.BlockSpec((Nb, 1, F), lambda i: (i, 0, 0)),
            pl.BlockSpec((Nb, 3, F), lambda i: (i, 0, 0)),
        ],
        out_shape=[
            jax.ShapeDtypeStruct((N, 1, F), jnp.float32),
            jax.ShapeDtypeStruct((N, 3, F), jnp.float32),
        ],
    )


def kernel(p1, p3, pair_i, pair_j, basis, diff, W_pp, b_pp, W_pi, W_ii,
           W_eq_pp, W_pix, W_out, b_out):
    N, _, F = p1.shape
    E = pair_i.shape[0]
    B = basis.shape[1]

    cat = jnp.concatenate([p1.reshape(N, F), p3.reshape(N, 3 * F)], axis=1)
    s = _make_gather(N, E, 4 * F)(cat, pair_i, pair_j)

    W_pi_perm = W_pi.reshape(F, F, B).transpose(0, 2, 1).reshape(F, F * B)
    i1f, ixf = _make_edge(E, F, B)(s, basis, diff, W_pi_perm, W_ii, W_pix)

    zeros = jnp.zeros((80, F), jnp.float32)
    acc1, acc3 = _make_scatter(N, E, F)(i1f, ixf, pair_i, zeros)

    p1t1, p3t1 = _make_node(N, F)(
        acc1, acc3, W_pp, b_pp.reshape(1, F), W_eq_pp, W_out,
        b_out.reshape(1, F))
    return (p1t1, p3t1, i1f.reshape(E, 1, F), ixf.reshape(E, 3, F))
